# Initial kernel scaffold; baseline (speedup 1.0000x reference)
import jax
import jax.numpy as jnp
from jax.experimental import pallas as pl


def kernel(x, W_switch, b_switch, W_experts, b_experts):
    raise NotImplementedError("write your pallas kernel here")


# speedup vs baseline: 2.5781x; 2.5781x over previous
import functools

import numpy as np
import jax
import jax.numpy as jnp
from jax.experimental import pallas as pl
from jax.experimental.pallas import tpu as pltpu

_CAPACITY_FACTOR = 1.2
_ROW_BLK = 256


def _bits_masked(k0, k1, n, n_max):
    ji = jnp.arange(n_max, dtype=jnp.int32)
    half = (n + 1) // 2
    x0 = ji.astype(jnp.uint32)
    x1 = jnp.where(ji < (n // 2), ji + half, 0).astype(jnp.uint32)
    ks2 = k0 ^ k1 ^ jnp.uint32(0x1BD11BDA)
    ks = (k0, k1, ks2)
    v0 = x0 + ks[0]
    v1 = x1 + ks[1]
    rotations = ((13, 15, 26, 6), (17, 29, 16, 24))
    for i in range(5):
        for r in rotations[i % 2]:
            v0 = v0 + v1
            v1 = (v1 << jnp.uint32(r)) | (v1 >> jnp.uint32(32 - r))
            v1 = v0 ^ v1
        v0 = v0 + ks[(i + 1) % 3]
        v1 = v1 + ks[(i + 2) % 3] + jnp.uint32(i + 1)
    lo = v1[jnp.clip(ji - half, 0, n_max - 1)]
    return jnp.where(ji < half, v0, lo)


def _round_key_data(E, max_rounds):
    keys = [jax.random.fold_in(jax.random.key(1), i) for i in range(E)]
    rows = []
    for _ in range(max_rounds):
        subs = []
        for i in range(E):
            keys[i], sub = jax.random.split(keys[i])
            subs.append(jax.random.key_data(sub))
        rows.append(jnp.stack(subs))
    return jnp.stack(rows)


def _expert_matmul_kernel(x_ref, w_ref, b_ref, o_ref):
    acc = jax.lax.dot_general(
        x_ref[0], w_ref[0],
        dimension_numbers=(((1,), (1,)), ((), ())),
        preferred_element_type=jnp.float32,
    )
    o_ref[0] = acc + b_ref[0]


def _expert_matmul(xg, W_experts, b_experts, cap_pad):
    E, D = W_experts.shape[0], W_experts.shape[1]
    grid = (E, cap_pad // _ROW_BLK)
    return pl.pallas_call(
        _expert_matmul_kernel,
        grid=grid,
        in_specs=[
            pl.BlockSpec((1, _ROW_BLK, D), lambda i, c: (i, c, 0)),
            pl.BlockSpec((1, D, D), lambda i, c: (i, 0, 0)),
            pl.BlockSpec((1, 1, D), lambda i, c: (i, 0, 0)),
        ],
        out_specs=pl.BlockSpec((1, _ROW_BLK, D), lambda i, c: (i, c, 0)),
        out_shape=jax.ShapeDtypeStruct((E, cap_pad, D), jnp.float32),
    )(xg.reshape(E, cap_pad, D), W_experts, b_experts.reshape(E, 1, D))


def kernel(x, W_switch, b_switch, W_experts, b_experts):
    b, s, d = x.shape
    E = W_switch.shape[0]
    n_tok = b * s
    xf = x.reshape(-1, d)

    logits = xf @ W_switch.T + b_switch
    probs = jax.nn.softmax(logits, axis=-1)
    route_probs = jnp.max(probs, axis=-1)
    routes = jnp.argmax(probs, axis=-1).astype(jnp.int32)

    capacity = int(_CAPACITY_FACTOR * n_tok / E)
    cap_pad = ((capacity + _ROW_BLK - 1) // _ROW_BLK) * _ROW_BLK
    rounds_np = np.array([int(np.ceil(3 * np.log(max(1, t)) / np.log(2)))
                          for t in range(n_tok + 1)], dtype=np.int32)
    rounds_table = jnp.asarray(rounds_np)
    max_rounds = int(rounds_np.max())

    eids = jnp.arange(E, dtype=jnp.int32)
    counts = jnp.sum(routes[None, :] == eids[:, None], axis=1).astype(jnp.int32)
    num_rounds = rounds_table[counts]
    r_needed = jnp.max(num_rounds)

    rk = _round_key_data(E, max_rounds)
    ji = jnp.arange(n_tok, dtype=jnp.int32)

    def round_body(carry):
        r, perm = carry
        kd = rk[r]
        bits = jax.vmap(lambda a, c, n: _bits_masked(a, c, n, n_tok))(
            kd[:, 0], kd[:, 1], counts)
        keyv = jnp.where(ji[None, :] < counts[:, None], bits,
                         jnp.uint32(0xFFFFFFFF))
        _, perm_r = jax.lax.sort((keyv, perm), dimension=1, num_keys=1,
                                 is_stable=True)
        perm = jnp.where((r < num_rounds)[:, None], perm_r, perm)
        return r + 1, perm

    perm0 = jnp.broadcast_to(ji[None, :], (E, n_tok)).astype(jnp.int32)
    _, perm = jax.lax.while_loop(lambda c: c[0] < r_needed, round_body,
                                 (jnp.int32(0), perm0))

    rowi = jnp.broadcast_to(eids[:, None], (E, n_tok))
    colj = jnp.broadcast_to(ji[None, :], (E, n_tok))
    inv = jnp.zeros((E, n_tok), jnp.int32).at[rowi, perm].set(colj)
    keep_rank = (counts[:, None] <= capacity) | (inv < capacity)

    sorted_tok = jnp.argsort(routes, stable=True).astype(jnp.int32)
    e_sorted = routes[sorted_tok]
    starts = jnp.concatenate([jnp.zeros((1,), jnp.int32),
                              jnp.cumsum(counts)[:-1].astype(jnp.int32)])
    pos = ji - starts[e_sorted]
    kept_sorted = keep_rank[e_sorted, pos]

    kept_count = jnp.minimum(counts, capacity)
    kept_before = jnp.concatenate([jnp.zeros((1,), jnp.int32),
                                   jnp.cumsum(kept_count)[:-1].astype(jnp.int32)])
    kc = jnp.cumsum(kept_sorted.astype(jnp.int32))
    slot = e_sorted * cap_pad + (kc - 1 - kept_before[e_sorted])

    d_flat = jnp.full((E * cap_pad,), n_tok, jnp.int32)
    d_flat = d_flat.at[jnp.where(kept_sorted, slot, E * cap_pad)].set(
        sorted_tok, mode="drop")
    g = jnp.zeros((n_tok,), jnp.int32).at[sorted_tok].set(
        jnp.where(kept_sorted, slot, E * cap_pad + sorted_tok))

    xf_pad = jnp.concatenate([xf, jnp.zeros((1, d), xf.dtype)], axis=0)
    xg = xf_pad[d_flat]
    yg = _expert_matmul(xg, W_experts, b_experts, cap_pad).reshape(-1, d)
    table = jnp.concatenate([yg, xf], axis=0)
    out = table[g] * route_probs[:, None]
    return out.reshape(b, s, d)
